# Initial kernel scaffold; baseline (speedup 1.0000x reference)
import jax
import jax.numpy as jnp
from jax.experimental import pallas as pl


def kernel(x, finer_grid_map):
    raise NotImplementedError("write your pallas kernel here")


# speedup vs baseline: 3.0438x; 3.0438x over previous
import functools

import jax
import jax.numpy as jnp
from jax import lax
from jax.experimental import pallas as pl
from jax.experimental.pallas import tpu as pltpu
from jax.experimental.pallas import tpu_sc as plsc

D = 256
B = 163842
NW = 32
CH = 128
BPW = 5120
NCH = BPW // CH
TAIL_BASE = NW * BPW
IDX_PAD = TAIL_BASE + 16


def _make_sc_gather():
    mesh = plsc.VectorSubcoreMesh(core_axis_name="c", subcore_axis_name="s")

    @functools.partial(
        pl.kernel,
        mesh=mesh,
        out_type=jax.ShapeDtypeStruct((B, D), jnp.float32),
        scratch_types=[
            pltpu.VMEM((BPW,), jnp.int32),
            pltpu.VMEM((CH, D), jnp.float32),
            pltpu.SemaphoreType.DMA,
        ],
    )
    def gather_kernel(x_hbm, idx_hbm, out_hbm, idx_v, rows_v, g_sem):
        cid = lax.axis_index("c")
        sid = lax.axis_index("s")
        wid = sid * 2 + cid
        base = wid * BPW

        pltpu.sync_copy(idx_hbm.at[pl.ds(base, BPW)], idx_v)

        def body(i, _):
            off = pl.multiple_of(i * CH, CH)
            pltpu.async_copy(x_hbm.at[idx_v.at[pl.ds(off, CH)]], rows_v,
                             g_sem).wait()
            pltpu.sync_copy(rows_v, out_hbm.at[pl.ds(base + off, CH)])
            return 0

        lax.fori_loop(0, NCH, body, 0)

        @pl.when(wid == 0)
        def _():
            pltpu.sync_copy(idx_hbm.at[pl.ds(TAIL_BASE, 16)],
                            idx_v.at[pl.ds(0, 16)])
            pltpu.async_copy(x_hbm.at[idx_v.at[pl.ds(0, 16)]],
                             rows_v.at[pl.ds(0, 16)], g_sem).wait()
            pltpu.sync_copy(rows_v.at[pl.ds(0, 2)],
                            out_hbm.at[pl.ds(TAIL_BASE, 2)])

    return gather_kernel


_gather = _make_sc_gather()


@jax.jit
def kernel(x, finer_grid_map):
    idx = jnp.pad(finer_grid_map, (0, IDX_PAD - B))
    return _gather(x, idx)


# speedup vs baseline: 3.5686x; 1.1724x over previous
import functools

import jax
import jax.numpy as jnp
from jax import lax
from jax.experimental import pallas as pl
from jax.experimental.pallas import tpu as pltpu
from jax.experimental.pallas import tpu_sc as plsc

D = 256
B = 163842
NW = 32
CH = 128
BPW = 5120
NCH = BPW // CH
TAIL_BASE = NW * BPW
IDX_PAD = TAIL_BASE + 16


def _make_sc_gather():
    mesh = plsc.VectorSubcoreMesh(core_axis_name="c", subcore_axis_name="s")

    @functools.partial(
        pl.kernel,
        mesh=mesh,
        out_type=jax.ShapeDtypeStruct((B, D), jnp.float32),
        scratch_types=[
            pltpu.VMEM((BPW,), jnp.int32),
            pltpu.VMEM((2, CH, D), jnp.float32),
            pltpu.SemaphoreType.DMA,
            pltpu.SemaphoreType.DMA,
        ],
    )
    def gather_kernel(x_hbm, idx_hbm, out_hbm, idx_v, rows_v, g_sem, s_sem):
        cid = lax.axis_index("c")
        sid = lax.axis_index("s")
        wid = sid * 2 + cid
        base = wid * BPW

        pltpu.sync_copy(idx_hbm.at[pl.ds(base, BPW)], idx_v)

        def gather_to(off, buf):
            return pltpu.async_copy(x_hbm.at[idx_v.at[pl.ds(off, CH)]],
                                    rows_v.at[buf], g_sem)

        gather_to(0, 0)

        G = NCH // 2

        def body(g, _):
            off0 = pl.multiple_of(g * (2 * CH), CH)
            off1 = off0 + CH
            pltpu.make_async_copy(x_hbm.at[idx_v.at[pl.ds(off0, CH)]],
                                  rows_v.at[0], g_sem).wait()
            gd1 = gather_to(off1, 1)
            sd0 = pltpu.async_copy(rows_v.at[0],
                                   out_hbm.at[pl.ds(base + off0, CH)], s_sem)
            gd1.wait()
            sd1 = pltpu.async_copy(rows_v.at[1],
                                   out_hbm.at[pl.ds(base + off1, CH)], s_sem)
            sd0.wait()

            @pl.when(g < G - 1)
            def _():
                gather_to(off1 + CH, 0)

            sd1.wait()
            return 0

        lax.fori_loop(0, G, body, 0)

        @pl.when(wid == 0)
        def _():
            pltpu.sync_copy(idx_hbm.at[pl.ds(TAIL_BASE, 16)],
                            idx_v.at[pl.ds(0, 16)])
            pltpu.async_copy(x_hbm.at[idx_v.at[pl.ds(0, 16)]],
                             rows_v.at[0, pl.ds(0, 16)], g_sem).wait()
            pltpu.sync_copy(rows_v.at[0, pl.ds(0, 2)],
                            out_hbm.at[pl.ds(TAIL_BASE, 2)])

    return gather_kernel


_gather = _make_sc_gather()


@jax.jit
def kernel(x, finer_grid_map):
    idx = jnp.pad(finer_grid_map, (0, IDX_PAD - B))
    return _gather(x, idx)
